# baseline (speedup 1.0000x reference)
import functools

import jax
import jax.numpy as jnp
from jax import lax
from jax.experimental import pallas as pl
from jax.experimental.pallas import tpu as pltpu
from jax.experimental.pallas import tpu_sc as plsc

TOKENS = 32768
GATE_DIM = 2048
HIDDEN_DIM = 256
NUM_EXPERTS = 64
TBLK = 2048

_SC_INFO = plsc.get_sparse_core_info()
_NC, _NS = _SC_INFO.num_cores, _SC_INFO.num_subcores
_NW = _NC * _NS
_SC_TILE = 256


def _mlp_body(x_ref, w1_ref, b1_ref, w2_ref, b2_ref, probs_ref, probst_ref):
    h = jax.lax.dot_general(x_ref[...], w1_ref[...], (((1,), (0,)), ((), ())),
                            preferred_element_type=jnp.float32)
    h = jnp.maximum(h + b1_ref[...], 0.0)
    logits = jax.lax.dot_general(h, w2_ref[...], (((1,), (0,)), ((), ())),
                                 preferred_element_type=jnp.float32)
    logits = logits + b2_ref[...]
    m = jnp.max(logits, axis=1, keepdims=True)
    e = jnp.exp(logits - m)
    s = jnp.sum(e, axis=1, keepdims=True)
    probs_ref[...] = e * (1.0 / s)

    logits_t = jax.lax.dot_general(w2_ref[...], h, (((0,), (1,)), ((), ())),
                                   preferred_element_type=jnp.float32)
    logits_t = logits_t + b2_ref[...][:, None]
    mt = jnp.max(logits_t, axis=0, keepdims=True)
    et = jnp.exp(logits_t - mt)
    st = jnp.sum(et, axis=0, keepdims=True)
    probst_ref[...] = et * (1.0 / st)


def _mlp_probs(x, W1, b1, W2, b2, ctok, c, probs_buf=None):
    ntok = x.shape[0]
    nblk = ctok // TBLK
    blk0 = c * nblk
    in_specs = [
        pl.BlockSpec((TBLK, GATE_DIM), lambda i: (blk0 + i, 0)),
        pl.BlockSpec((GATE_DIM, HIDDEN_DIM), lambda i: (0, 0)),
        pl.BlockSpec((HIDDEN_DIM,), lambda i: (0,)),
        pl.BlockSpec((HIDDEN_DIM, NUM_EXPERTS), lambda i: (0, 0)),
        pl.BlockSpec((NUM_EXPERTS,), lambda i: (0,)),
    ]
    args = [x, W1, b1, W2, b2]
    aliases = {}
    if probs_buf is not None:
        in_specs.append(pl.BlockSpec(memory_space=pl.ANY))
        args.append(probs_buf)
        aliases = {5: 0}

    def body(*refs):
        if probs_buf is not None:
            refs = refs[:5] + refs[6:]
        _mlp_body(*refs)

    return pl.pallas_call(
        body,
        grid=(nblk,),
        in_specs=in_specs,
        out_specs=[
            pl.BlockSpec((TBLK, NUM_EXPERTS), lambda i: (blk0 + i, 0)),
            pl.BlockSpec((NUM_EXPERTS, TBLK), lambda i: (0, i)),
        ],
        out_shape=[
            jax.ShapeDtypeStruct((ntok, NUM_EXPERTS), jnp.float32),
            jax.ShapeDtypeStruct((NUM_EXPERTS, ctok), jnp.float32),
        ],
        input_output_aliases=aliases,
    )(*args)


def _sc_route_body(ntok, probst_hbm, i1_hbm, i2_hbm, p1_hbm, p2_hbm,
                   tile_a, tile_b, i1_v, i2_v, p1_v, p2_v, sem_a, sem_b):
    tok_per_w = ntok // _NW
    wid = lax.axis_index("s") * _NC + lax.axis_index("c")
    base = wid * tok_per_w
    nblk = tok_per_w // _SC_TILE
    tiles = [tile_a, tile_b]
    sems = [sem_a, sem_b]

    copies = [None] * nblk
    copies[0] = pltpu.async_copy(
        probst_hbm.at[:, pl.ds(base, _SC_TILE)], tiles[0], sems[0])
    for blk in range(nblk):
        if blk + 1 < nblk:
            copies[blk + 1] = pltpu.async_copy(
                probst_hbm.at[:, pl.ds(base + (blk + 1) * _SC_TILE, _SC_TILE)],
                tiles[(blk + 1) % 2], sems[(blk + 1) % 2])
        copies[blk].wait()
        tile_v = tiles[blk % 2]
        off = blk * _SC_TILE

        def body(g, _):
            t16 = g * 16
            val1 = tile_v[0, pl.ds(t16, 16)]
            idx1 = jnp.zeros((16,), jnp.int32)
            val2 = jnp.full((16,), -1.0, jnp.float32)
            idx2 = jnp.zeros((16,), jnp.int32)
            for e in range(1, NUM_EXPERTS):
                v = tile_v[e, pl.ds(t16, 16)]
                gt = v > val1
                gt2 = v > val2
                val2 = jnp.maximum(val2, jnp.minimum(val1, v))
                idx2 = jnp.where(gt, idx1, jnp.where(gt2, jnp.int32(e), idx2))
                val1 = jnp.maximum(val1, v)
                idx1 = jnp.where(gt, jnp.int32(e), idx1)
            i1_v[pl.ds(off + t16, 16)] = idx1
            i2_v[pl.ds(off + t16, 16)] = idx2
            p1_v[pl.ds(off + t16, 16)] = val1
            p2_v[pl.ds(off + t16, 16)] = val2
            return 0

        lax.fori_loop(0, _SC_TILE // 16, body, 0, unroll=2)

    pltpu.sync_copy(i1_v, i1_hbm.at[pl.ds(base, tok_per_w)])
    pltpu.sync_copy(i2_v, i2_hbm.at[pl.ds(base, tok_per_w)])
    pltpu.sync_copy(p1_v, p1_hbm.at[pl.ds(base, tok_per_w)])
    pltpu.sync_copy(p2_v, p2_hbm.at[pl.ds(base, tok_per_w)])


def _sc_route(probs_t):
    ntok = probs_t.shape[1]
    mesh = plsc.VectorSubcoreMesh(core_axis_name="c", subcore_axis_name="s")
    return pl.kernel(
        functools.partial(_sc_route_body, ntok),
        mesh=mesh,
        out_type=[
            jax.ShapeDtypeStruct((ntok,), jnp.int32),
            jax.ShapeDtypeStruct((ntok,), jnp.int32),
            jax.ShapeDtypeStruct((ntok,), jnp.float32),
            jax.ShapeDtypeStruct((ntok,), jnp.float32),
        ],
        scratch_types=[
            pltpu.VMEM((NUM_EXPERTS, _SC_TILE), jnp.float32),
            pltpu.VMEM((NUM_EXPERTS, _SC_TILE), jnp.float32),
            pltpu.VMEM((ntok // _NW,), jnp.int32),
            pltpu.VMEM((ntok // _NW,), jnp.int32),
            pltpu.VMEM((ntok // _NW,), jnp.float32),
            pltpu.VMEM((ntok // _NW,), jnp.float32),
            pltpu.SemaphoreType.DMA,
            pltpu.SemaphoreType.DMA,
        ],
    )(probs_t)


def kernel(gate_features, W1, b1, W2, b2):
    nchunk = 1
    ctok = TOKENS // nchunk
    probs = None
    parts = []
    for c in range(nchunk):
        probs, probst_c = _mlp_probs(gate_features, W1, b1, W2, b2, ctok, c,
                                     probs_buf=probs)
        parts.append(_sc_route(probst_c))
    i1 = jnp.concatenate([p[0] for p in parts], axis=0)
    i2 = jnp.concatenate([p[1] for p in parts], axis=0)
    p1 = jnp.concatenate([p[2] for p in parts], axis=0)
    p2 = jnp.concatenate([p[3] for p in parts], axis=0)
    topk_idx = jnp.stack([i1, i2], axis=1)
    topk_probs = jnp.stack([p1, p2], axis=1)
    return (i1, probs, topk_idx, topk_probs)


# speedup vs baseline: 1.0058x; 1.0058x over previous
import functools

import jax
import jax.numpy as jnp
from jax import lax
from jax.experimental import pallas as pl
from jax.experimental.pallas import tpu as pltpu
from jax.experimental.pallas import tpu_sc as plsc

TOKENS = 32768
GATE_DIM = 2048
HIDDEN_DIM = 256
NUM_EXPERTS = 64
TBLK = 2048

_SC_INFO = plsc.get_sparse_core_info()
_NC, _NS = _SC_INFO.num_cores, _SC_INFO.num_subcores
_NW = _NC * _NS
_SC_TILE = 256


def _mlp_body(x_ref, w1_ref, b1_ref, w2_ref, b2_ref, probs_ref, probst_ref):
    h = jax.lax.dot_general(x_ref[...], w1_ref[...], (((1,), (0,)), ((), ())),
                            preferred_element_type=jnp.float32)
    h = jnp.maximum(h + b1_ref[...], 0.0)
    logits = jax.lax.dot_general(h, w2_ref[...], (((1,), (0,)), ((), ())),
                                 preferred_element_type=jnp.float32)
    logits = logits + b2_ref[...]
    m = jnp.max(logits, axis=1, keepdims=True)
    e = jnp.exp(logits - m)
    s = jnp.sum(e, axis=1, keepdims=True)
    probs_ref[...] = e * (1.0 / s)

    logits_t = jax.lax.dot_general(w2_ref[...], h, (((0,), (1,)), ((), ())),
                                   preferred_element_type=jnp.float32)
    logits_t = logits_t + b2_ref[...][:, None]
    mt = jnp.max(logits_t, axis=0, keepdims=True)
    et = jnp.exp(logits_t - mt)
    st = jnp.sum(et, axis=0, keepdims=True)
    probst_ref[...] = et * (1.0 / st)


def _mlp_probs(x, W1, b1, W2, b2):
    ntok = x.shape[0]
    nblk = ntok // TBLK
    return pl.pallas_call(
        _mlp_body,
        grid=(nblk,),
        in_specs=[
            pl.BlockSpec((TBLK, GATE_DIM), lambda i: (i, 0)),
            pl.BlockSpec((GATE_DIM, HIDDEN_DIM), lambda i: (0, 0)),
            pl.BlockSpec((HIDDEN_DIM,), lambda i: (0,)),
            pl.BlockSpec((HIDDEN_DIM, NUM_EXPERTS), lambda i: (0, 0)),
            pl.BlockSpec((NUM_EXPERTS,), lambda i: (0,)),
        ],
        out_specs=[
            pl.BlockSpec((TBLK, NUM_EXPERTS), lambda i: (i, 0)),
            pl.BlockSpec((NUM_EXPERTS, TBLK), lambda i: (0, i)),
        ],
        out_shape=[
            jax.ShapeDtypeStruct((ntok, NUM_EXPERTS), jnp.float32),
            jax.ShapeDtypeStruct((NUM_EXPERTS, ntok), jnp.float32),
        ],
    )(x, W1, b1, W2, b2)


def _sc_route_body(ntok, probst_hbm, i1_hbm, i2_hbm, p1_hbm, p2_hbm,
                   tile_a, tile_b, i1_v, i2_v, p1_v, p2_v, sem_a, sem_b):
    tok_per_w = ntok // _NW
    wid = lax.axis_index("s") * _NC + lax.axis_index("c")
    base = wid * tok_per_w
    nblk = tok_per_w // _SC_TILE
    tiles = [tile_a, tile_b]
    sems = [sem_a, sem_b]

    copies = [None] * nblk
    copies[0] = pltpu.async_copy(
        probst_hbm.at[:, pl.ds(base, _SC_TILE)], tiles[0], sems[0])
    for blk in range(nblk):
        if blk + 1 < nblk:
            copies[blk + 1] = pltpu.async_copy(
                probst_hbm.at[:, pl.ds(base + (blk + 1) * _SC_TILE, _SC_TILE)],
                tiles[(blk + 1) % 2], sems[(blk + 1) % 2])
        copies[blk].wait()
        tile_v = tiles[blk % 2]
        off = blk * _SC_TILE

        def body(g, _):
            t16 = g * 16
            val1 = tile_v[0, pl.ds(t16, 16)]
            idx1 = jnp.zeros((16,), jnp.int32)
            val2 = jnp.full((16,), -1.0, jnp.float32)
            idx2 = jnp.zeros((16,), jnp.int32)
            for e in range(1, NUM_EXPERTS):
                v = tile_v[e, pl.ds(t16, 16)]
                gt = v > val1
                gt2 = v > val2
                val2 = jnp.maximum(val2, jnp.minimum(val1, v))
                idx2 = jnp.where(gt, idx1, jnp.where(gt2, jnp.int32(e), idx2))
                val1 = jnp.maximum(val1, v)
                idx1 = jnp.where(gt, jnp.int32(e), idx1)
            i1_v[pl.ds(off + t16, 16)] = idx1
            i2_v[pl.ds(off + t16, 16)] = idx2
            p1_v[pl.ds(off + t16, 16)] = val1
            p2_v[pl.ds(off + t16, 16)] = val2
            return 0

        lax.fori_loop(0, _SC_TILE // 16, body, 0)

    pltpu.sync_copy(i1_v, i1_hbm.at[pl.ds(base, tok_per_w)])
    pltpu.sync_copy(i2_v, i2_hbm.at[pl.ds(base, tok_per_w)])
    pltpu.sync_copy(p1_v, p1_hbm.at[pl.ds(base, tok_per_w)])
    pltpu.sync_copy(p2_v, p2_hbm.at[pl.ds(base, tok_per_w)])


def _sc_route(probs_t):
    ntok = probs_t.shape[1]
    mesh = plsc.VectorSubcoreMesh(core_axis_name="c", subcore_axis_name="s")
    return pl.kernel(
        functools.partial(_sc_route_body, ntok),
        mesh=mesh,
        out_type=[
            jax.ShapeDtypeStruct((ntok,), jnp.int32),
            jax.ShapeDtypeStruct((ntok,), jnp.int32),
            jax.ShapeDtypeStruct((ntok,), jnp.float32),
            jax.ShapeDtypeStruct((ntok,), jnp.float32),
        ],
        scratch_types=[
            pltpu.VMEM((NUM_EXPERTS, _SC_TILE), jnp.float32),
            pltpu.VMEM((NUM_EXPERTS, _SC_TILE), jnp.float32),
            pltpu.VMEM((ntok // _NW,), jnp.int32),
            pltpu.VMEM((ntok // _NW,), jnp.int32),
            pltpu.VMEM((ntok // _NW,), jnp.float32),
            pltpu.VMEM((ntok // _NW,), jnp.float32),
            pltpu.SemaphoreType.DMA,
            pltpu.SemaphoreType.DMA,
        ],
    )(probs_t)


def kernel(gate_features, W1, b1, W2, b2):
    probs, probs_t = _mlp_probs(gate_features, W1, b1, W2, b2)
    i1, i2, p1, p2 = _sc_route(probs_t)
    topk_idx = jnp.stack([i1, i2], axis=1)
    topk_probs = jnp.stack([p1, p2], axis=1)
    return (i1, probs, topk_idx, topk_probs)
